# baseline (speedup 1.0000x reference)
import math

import jax
import jax.numpy as jnp
from jax import lax
from jax.experimental import pallas as pl
from jax.experimental.pallas import tpu as pltpu
from jax.experimental.pallas import tpu_sc as plsc

MODEL_DIM = 1024
PROJ_DIM = 4
NUM_EXPERTS = 64
COMPONENTS = 4
TC = NUM_EXPERTS * COMPONENTS
S_TOTAL = 16384
MM_BLOCK = 4096

NC = 2
NS = 16
NW = NC * NS
L = 16
TPT = S_TOTAL // NW
SUB = 128

LN2 = 0.6931471805599453


def _proj_kernel(x_ref, w_ref, o_ref):
    o_ref[...] = jnp.dot(x_ref[...], w_ref[...],
                         preferred_element_type=jnp.float32)


def _ln_scalar(z):
    zb = lax.bitcast_convert_type(z, jnp.int32)
    ex = lax.shift_right_logical(zb, 23) - 127
    mant = lax.bitcast_convert_type(
        lax.bitwise_or(lax.bitwise_and(zb, 0x7FFFFF), 127 << 23),
        jnp.float32)
    big = mant > 1.4142135623730951
    mant = jnp.where(big, mant * 0.5, mant)
    exf = jnp.where(big, ex + 1, ex).astype(jnp.float32)
    t = mant - 1.0
    p = -1.0 / 10.0
    for k in range(9, 0, -1):
        p = ((1.0 if k % 2 == 1 else -1.0) / k) + t * p
    return exf * LN2 + t * p


def _scalar_tree(vec, op):
    xs = [vec[i] for i in range(L)]
    while len(xs) > 1:
        xs = [op(xs[i], xs[i + 1]) for i in range(0, len(xs), 2)]
    return xs[0]


def _sc_gmm(proj_hbm, params_hbm, post_hbm, exp_hbm, nll_hbm,
            proj_v, qsq_v, params_v, logit_v, post_v, exp_v, nll_v):
    cid = lax.axis_index("c")
    sid = lax.axis_index("s")
    wid = sid * NC + cid
    base = wid * TPT
    npar = 2 * PROJ_DIM * TC + TC

    pltpu.sync_copy(params_hbm, params_v)
    pltpu.sync_copy(proj_hbm.at[pl.ds(base * PROJ_DIM, TPT * PROJ_DIM)],
                    proj_v)

    def sq_body(i, c):
        v = proj_v[pl.ds(i * L, L)]
        qsq_v[pl.ds(i * L, L)] = v * v
        return c
    lax.fori_loop(0, TPT * PROJ_DIM // L, sq_body, 0)

    acc = jnp.zeros((L,), jnp.float32)

    def chunk_body(sc_i, acc):
        t0 = sc_i * SUB

        def p1_grp(j, c):
            bias_v = params_v[pl.ds(2 * PROJ_DIM * TC + j * L, L)]
            avs = [params_v[pl.ds(p * TC + j * L, L)] for p in range(PROJ_DIM)]
            bvs = [params_v[pl.ds((PROJ_DIM + p) * TC + j * L, L)]
                   for p in range(PROJ_DIM)]

            def p1_body(tb, c2):
                off = (t0 + tb * 4) * PROJ_DIM
                pv = proj_v[pl.ds(off, L)]
                qv = qsq_v[pl.ds(off, L)]
                for tt in range(4):
                    v = bias_v
                    for p in range(PROJ_DIM):
                        v = (v + avs[p] * pv[tt * PROJ_DIM + p]
                             + bvs[p] * qv[tt * PROJ_DIM + p])
                    logit_v[pl.ds((tb * 4 + tt) * TC + j * L, L)] = v
                return c2
            return lax.fori_loop(0, SUB // 4, p1_body, c)
        lax.fori_loop(0, TC // L, p1_grp, 0)

        def p1b_grp(r, c):
            b2 = [params_v[pl.ds(npar + 2 * PROJ_DIM * TC + (k * 4 + r) * L,
                                 L)] for k in range(COMPONENTS)]
            a2 = [[params_v[pl.ds(npar + p * TC + (k * 4 + r) * L, L)]
                   for p in range(PROJ_DIM)] for k in range(COMPONENTS)]
            c2 = [[params_v[pl.ds(npar + (PROJ_DIM + p) * TC + (k * 4 + r) * L,
                                  L)] for p in range(PROJ_DIM)]
                  for k in range(COMPONENTS)]

            def p1b_body(tb, cc):
                off = (t0 + tb * 4) * PROJ_DIM
                pv = proj_v[pl.ds(off, L)]
                qv = qsq_v[pl.ds(off, L)]
                for tt in range(4):
                    em = None
                    for k in range(COMPONENTS):
                        v = b2[k]
                        for p in range(PROJ_DIM):
                            v = (v + a2[k][p] * pv[tt * PROJ_DIM + p]
                                 + c2[k][p] * qv[tt * PROJ_DIM + p])
                        em = v if em is None else jnp.maximum(em, v)
                    exp_v[pl.ds((tb * 4 + tt) * NUM_EXPERTS + r * L, L)] = em
                return cc
            return lax.fori_loop(0, SUB // 4, p1b_body, c)
        lax.fori_loop(0, 4, p1b_grp, 0)

        def p2_body(t, a):
            offl = t * TC
            lv = [logit_v[pl.ds(offl + j * L, L)] for j in range(16)]
            mv = lv[0]
            for j in range(1, 16):
                mv = jnp.maximum(mv, lv[j])
            m = _scalar_tree(mv, jnp.maximum)
            evs = [jnp.exp(lv[j] - m) for j in range(16)]
            sv = evs[0]
            for j in range(1, 16):
                sv = sv + evs[j]
            z = _scalar_tree(sv, lambda x, y: x + y)
            lnz = _ln_scalar(z)
            invz = jnp.exp(jnp.zeros((L,), jnp.float32) - lnz)
            for j in range(16):
                post_v[pl.ds(offl + j * L, L)] = evs[j] * invz
            offe = t * NUM_EXPERTS
            for r in range(4):
                em = exp_v[pl.ds(offe + r * L, L)]
                exp_v[pl.ds(offe + r * L, L)] = jnp.exp(em - m) * invz
            return a + (m + lnz)
        acc = lax.fori_loop(0, SUB, p2_body, acc)

        pltpu.sync_copy(post_v,
                        post_hbm.at[pl.ds((base + t0) * TC, SUB * TC)])
        pltpu.sync_copy(exp_v,
                        exp_hbm.at[pl.ds((base + t0) * NUM_EXPERTS,
                                         SUB * NUM_EXPERTS)])
        return acc

    acc = lax.fori_loop(0, TPT // SUB, chunk_body, acc)

    nll_v[pl.ds(0, L)] = acc * (1.0 / L)
    pltpu.sync_copy(nll_v, nll_hbm.at[pl.ds(wid * L, L)])


def kernel(input, W_proj, means, log_vars, mix_logits):
    S = input.shape[0]

    mix_prob = jax.nn.softmax(jax.lax.stop_gradient(mix_logits))
    drop_mask = jax.random.bernoulli(jax.random.key(42), mix_prob)
    log_mix = jax.nn.log_softmax(mix_logits)
    vars_ = jnp.exp(log_vars)
    inv_v = 1.0 / (vars_ + 1e-06)
    log_det = jnp.sum(log_vars, axis=-1)
    bias0 = log_mix - 0.5 * (log_det + PROJ_DIM * math.log(2 * math.pi)
                             + jnp.sum(means * means * inv_v, axis=-1))
    bias0 = jnp.where(drop_mask, -1e30, bias0)
    a0 = (means * inv_v).T
    b0 = (-0.5 * inv_v).T
    i_all = jnp.arange(TC)
    idx_g = (i_all % NUM_EXPERTS) * COMPONENTS + i_all // NUM_EXPERTS
    params = jnp.concatenate(
        [a0.reshape(-1), b0.reshape(-1), bias0,
         a0[:, idx_g].reshape(-1), b0[:, idx_g].reshape(-1),
         bias0[idx_g]]).astype(jnp.float32)

    proj = pl.pallas_call(
        _proj_kernel,
        grid=(S // MM_BLOCK,),
        in_specs=[
            pl.BlockSpec((MM_BLOCK, MODEL_DIM), lambda i: (i, 0)),
            pl.BlockSpec((MODEL_DIM, PROJ_DIM), lambda i: (0, 0)),
        ],
        out_specs=pl.BlockSpec((MM_BLOCK, PROJ_DIM), lambda i: (i, 0)),
        out_shape=jax.ShapeDtypeStruct((S, PROJ_DIM), jnp.float32),
        compiler_params=pltpu.CompilerParams(
            dimension_semantics=("parallel",),
        ),
    )(input, W_proj)

    sc_fn = pl.kernel(
        _sc_gmm,
        out_type=[
            jax.ShapeDtypeStruct((S * TC,), jnp.float32),
            jax.ShapeDtypeStruct((S * NUM_EXPERTS,), jnp.float32),
            jax.ShapeDtypeStruct((NW * L,), jnp.float32),
        ],
        mesh=plsc.VectorSubcoreMesh(core_axis_name="c",
                                    subcore_axis_name="s"),
        scratch_types=[
            pltpu.VMEM((TPT * PROJ_DIM,), jnp.float32),
            pltpu.VMEM((TPT * PROJ_DIM,), jnp.float32),
            pltpu.VMEM((2 * (2 * PROJ_DIM * TC + TC),), jnp.float32),
            pltpu.VMEM((SUB * TC,), jnp.float32),
            pltpu.VMEM((SUB * TC,), jnp.float32),
            pltpu.VMEM((SUB * NUM_EXPERTS,), jnp.float32),
            pltpu.VMEM((L,), jnp.float32),
        ],
    )
    post_f, exp_f, nll_parts = sc_fn(proj.reshape(-1), params)

    nll = -(jnp.sum(nll_parts) / S)
    return (exp_f.reshape(S, NUM_EXPERTS), post_f.reshape(S, TC), nll)


# speedup vs baseline: 1.0551x; 1.0551x over previous
import math

import jax
import jax.numpy as jnp
from jax import lax
from jax.experimental import pallas as pl
from jax.experimental.pallas import tpu as pltpu
from jax.experimental.pallas import tpu_sc as plsc

MODEL_DIM = 1024
PROJ_DIM = 4
NUM_EXPERTS = 64
COMPONENTS = 4
TC = NUM_EXPERTS * COMPONENTS
S_TOTAL = 16384
MM_BLOCK = 4096

NC = 2
NS = 16
NW = NC * NS
L = 16
TPT = S_TOTAL // NW
SUB = 128

LN2 = 0.6931471805599453


def _proj_kernel(x_ref, w_ref, o_ref):
    o_ref[...] = jnp.dot(x_ref[...], w_ref[...],
                         preferred_element_type=jnp.float32)


def _ln_scalar(z):
    zb = lax.bitcast_convert_type(z, jnp.int32)
    ex = lax.shift_right_logical(zb, 23) - 127
    mant = lax.bitcast_convert_type(
        lax.bitwise_or(lax.bitwise_and(zb, 0x7FFFFF), 127 << 23),
        jnp.float32)
    big = mant > 1.4142135623730951
    mant = jnp.where(big, mant * 0.5, mant)
    exf = jnp.where(big, ex + 1, ex).astype(jnp.float32)
    t = mant - 1.0
    p = -1.0 / 10.0
    for k in range(9, 0, -1):
        p = ((1.0 if k % 2 == 1 else -1.0) / k) + t * p
    return exf * LN2 + t * p


def _scalar_tree(vec, op):
    xs = [vec[i] for i in range(L)]
    while len(xs) > 1:
        xs = [op(xs[i], xs[i + 1]) for i in range(0, len(xs), 2)]
    return xs[0]


def _sc_gmm(proj_hbm, params_hbm, post_hbm, exp_hbm, nll_hbm,
            proj_v, qsq_v, params_v, logit_v, post_v, exp_v, nll_v):
    cid = lax.axis_index("c")
    sid = lax.axis_index("s")
    wid = sid * NC + cid
    base = wid * TPT
    npar = 2 * PROJ_DIM * TC + TC

    pltpu.sync_copy(params_hbm, params_v)
    pltpu.sync_copy(proj_hbm.at[pl.ds(base * PROJ_DIM, TPT * PROJ_DIM)],
                    proj_v)

    def sq_body(i):
        v = proj_v[pl.ds(i * L, L)]
        qsq_v[pl.ds(i * L, L)] = v * v
    plsc.parallel_loop(0, TPT * PROJ_DIM // L, unroll=4)(sq_body)

    acc = jnp.zeros((L,), jnp.float32)

    def chunk_body(sc_i, acc):
        t0 = sc_i * SUB

        def p1_grp(j, c):
            bias_v = params_v[pl.ds(2 * PROJ_DIM * TC + j * L, L)]
            avs = [params_v[pl.ds(p * TC + j * L, L)] for p in range(PROJ_DIM)]
            bvs = [params_v[pl.ds((PROJ_DIM + p) * TC + j * L, L)]
                   for p in range(PROJ_DIM)]

            def p1_body(tb):
                off = (t0 + tb * 4) * PROJ_DIM
                pv = proj_v[pl.ds(off, L)]
                qv = qsq_v[pl.ds(off, L)]
                for tt in range(4):
                    v = bias_v
                    for p in range(PROJ_DIM):
                        v = (v + avs[p] * pv[tt * PROJ_DIM + p]
                             + bvs[p] * qv[tt * PROJ_DIM + p])
                    logit_v[pl.ds((tb * 4 + tt) * TC + j * L, L)] = v
            plsc.parallel_loop(0, SUB // 4, unroll=2)(p1_body)
            return c
        lax.fori_loop(0, TC // L, p1_grp, 0)

        def p1b_grp(r, c):
            b2 = [params_v[pl.ds(npar + 2 * PROJ_DIM * TC + (k * 4 + r) * L,
                                 L)] for k in range(COMPONENTS)]
            a2 = [[params_v[pl.ds(npar + p * TC + (k * 4 + r) * L, L)]
                   for p in range(PROJ_DIM)] for k in range(COMPONENTS)]
            c2 = [[params_v[pl.ds(npar + (PROJ_DIM + p) * TC + (k * 4 + r) * L,
                                  L)] for p in range(PROJ_DIM)]
                  for k in range(COMPONENTS)]

            def p1b_body(tb):
                off = (t0 + tb * 4) * PROJ_DIM
                pv = proj_v[pl.ds(off, L)]
                qv = qsq_v[pl.ds(off, L)]
                for tt in range(4):
                    em = None
                    for k in range(COMPONENTS):
                        v = b2[k]
                        for p in range(PROJ_DIM):
                            v = (v + a2[k][p] * pv[tt * PROJ_DIM + p]
                                 + c2[k][p] * qv[tt * PROJ_DIM + p])
                        em = v if em is None else jnp.maximum(em, v)
                    exp_v[pl.ds((tb * 4 + tt) * NUM_EXPERTS + r * L, L)] = em
            plsc.parallel_loop(0, SUB // 4, unroll=2)(p1b_body)
            return c
        lax.fori_loop(0, 4, p1b_grp, 0)

        def p2_body(t, a):
            offl = t * TC
            lv = [logit_v[pl.ds(offl + j * L, L)] for j in range(16)]
            mv = lv[0]
            for j in range(1, 16):
                mv = jnp.maximum(mv, lv[j])
            m = _scalar_tree(mv, jnp.maximum)
            evs = [jnp.exp(lv[j] - m) for j in range(16)]
            sv = evs[0]
            for j in range(1, 16):
                sv = sv + evs[j]
            z = _scalar_tree(sv, lambda x, y: x + y)
            lnz = _ln_scalar(z)
            invz = jnp.exp(jnp.zeros((L,), jnp.float32) - lnz)
            for j in range(16):
                post_v[pl.ds(offl + j * L, L)] = evs[j] * invz
            offe = t * NUM_EXPERTS
            for r in range(4):
                em = exp_v[pl.ds(offe + r * L, L)]
                exp_v[pl.ds(offe + r * L, L)] = jnp.exp(em - m) * invz
            return a + (m + lnz)
        acc = plsc.parallel_loop(0, SUB, unroll=2, carry=acc)(p2_body)

        pltpu.sync_copy(post_v,
                        post_hbm.at[pl.ds((base + t0) * TC, SUB * TC)])
        pltpu.sync_copy(exp_v,
                        exp_hbm.at[pl.ds((base + t0) * NUM_EXPERTS,
                                         SUB * NUM_EXPERTS)])
        return acc

    acc = lax.fori_loop(0, TPT // SUB, chunk_body, acc)

    nll_v[pl.ds(0, L)] = acc * (1.0 / L)
    pltpu.sync_copy(nll_v, nll_hbm.at[pl.ds(wid * L, L)])


def kernel(input, W_proj, means, log_vars, mix_logits):
    S = input.shape[0]

    mix_prob = jax.nn.softmax(jax.lax.stop_gradient(mix_logits))
    drop_mask = jax.random.bernoulli(jax.random.key(42), mix_prob)
    log_mix = jax.nn.log_softmax(mix_logits)
    vars_ = jnp.exp(log_vars)
    inv_v = 1.0 / (vars_ + 1e-06)
    log_det = jnp.sum(log_vars, axis=-1)
    bias0 = log_mix - 0.5 * (log_det + PROJ_DIM * math.log(2 * math.pi)
                             + jnp.sum(means * means * inv_v, axis=-1))
    bias0 = jnp.where(drop_mask, -1e30, bias0)
    a0 = (means * inv_v).T
    b0 = (-0.5 * inv_v).T
    i_all = jnp.arange(TC)
    idx_g = (i_all % NUM_EXPERTS) * COMPONENTS + i_all // NUM_EXPERTS
    params = jnp.concatenate(
        [a0.reshape(-1), b0.reshape(-1), bias0,
         a0[:, idx_g].reshape(-1), b0[:, idx_g].reshape(-1),
         bias0[idx_g]]).astype(jnp.float32)

    proj = pl.pallas_call(
        _proj_kernel,
        grid=(S // MM_BLOCK,),
        in_specs=[
            pl.BlockSpec((MM_BLOCK, MODEL_DIM), lambda i: (i, 0)),
            pl.BlockSpec((MODEL_DIM, PROJ_DIM), lambda i: (0, 0)),
        ],
        out_specs=pl.BlockSpec((MM_BLOCK, PROJ_DIM), lambda i: (i, 0)),
        out_shape=jax.ShapeDtypeStruct((S, PROJ_DIM), jnp.float32),
        compiler_params=pltpu.CompilerParams(
            dimension_semantics=("parallel",),
        ),
    )(input, W_proj)

    sc_fn = pl.kernel(
        _sc_gmm,
        out_type=[
            jax.ShapeDtypeStruct((S * TC,), jnp.float32),
            jax.ShapeDtypeStruct((S * NUM_EXPERTS,), jnp.float32),
            jax.ShapeDtypeStruct((NW * L,), jnp.float32),
        ],
        mesh=plsc.VectorSubcoreMesh(core_axis_name="c",
                                    subcore_axis_name="s"),
        scratch_types=[
            pltpu.VMEM((TPT * PROJ_DIM,), jnp.float32),
            pltpu.VMEM((TPT * PROJ_DIM,), jnp.float32),
            pltpu.VMEM((2 * (2 * PROJ_DIM * TC + TC),), jnp.float32),
            pltpu.VMEM((SUB * TC,), jnp.float32),
            pltpu.VMEM((SUB * TC,), jnp.float32),
            pltpu.VMEM((SUB * NUM_EXPERTS,), jnp.float32),
            pltpu.VMEM((L,), jnp.float32),
        ],
    )
    post_f, exp_f, nll_parts = sc_fn(proj.reshape(-1), params)

    nll = -(jnp.sum(nll_parts) / S)
    return (exp_f.reshape(S, NUM_EXPERTS), post_f.reshape(S, TC), nll)


# speedup vs baseline: 4.1524x; 3.9356x over previous
import math

import jax
import jax.numpy as jnp
from jax.experimental import pallas as pl
from jax.experimental.pallas import tpu as pltpu

MODEL_DIM = 1024
PROJ_DIM = 4
NUM_EXPERTS = 64
COMPONENTS = 4
TC = NUM_EXPERTS * COMPONENTS
BLOCK_S = 4096


def _gmm_kernel(x_ref, w_ref, a_ref, b_ref, bias_ref, post_ref, exp_ref,
                nll_ref):
    x = x_ref[...]
    proj = jnp.dot(x, w_ref[...], preferred_element_type=jnp.float32)
    psq = proj * proj
    lg = (jnp.dot(proj, a_ref[...], preferred_element_type=jnp.float32)
          + jnp.dot(psq, b_ref[...], preferred_element_type=jnp.float32)
          + bias_ref[...])

    logits = lg[:, :TC]
    m = jnp.max(logits, axis=-1, keepdims=True)
    e = jnp.exp(logits - m)
    z = jnp.sum(e, axis=-1, keepdims=True)
    inv_z = 1.0 / z
    post_ref[...] = e * inv_z

    g0 = jnp.maximum(lg[:, TC:TC + NUM_EXPERTS],
                     lg[:, TC + NUM_EXPERTS:TC + 2 * NUM_EXPERTS])
    g1 = jnp.maximum(lg[:, TC + 2 * NUM_EXPERTS:TC + 3 * NUM_EXPERTS],
                     lg[:, TC + 3 * NUM_EXPERTS:])
    gmax = jnp.maximum(g0, g1)
    exp_ref[...] = jnp.exp(gmax - m) * inv_z

    s = jnp.sum(m[:, 0] + jnp.log(z[:, 0]))
    nll_ref[...] = jnp.broadcast_to(s, (1, 1, 128))


def kernel(input, W_proj, means, log_vars, mix_logits):
    S = input.shape[0]
    n_blocks = S // BLOCK_S

    mix_prob = jax.nn.softmax(jax.lax.stop_gradient(mix_logits))
    drop_mask = jax.random.bernoulli(jax.random.key(42), mix_prob)
    log_mix = jax.nn.log_softmax(mix_logits)
    vars_ = jnp.exp(log_vars)
    inv_v = 1.0 / (vars_ + 1e-06)
    log_det = jnp.sum(log_vars, axis=-1)
    bias0 = log_mix - 0.5 * (log_det + PROJ_DIM * math.log(2 * math.pi)
                             + jnp.sum(means * means * inv_v, axis=-1))
    bias0 = jnp.where(drop_mask, -1e30, bias0)

    a0 = (means * inv_v).T
    b0 = (-0.5 * inv_v).T
    c = jnp.arange(TC)
    perm = (c % COMPONENTS) * NUM_EXPERTS + c // COMPONENTS
    inv_perm = jnp.argsort(perm)
    A = jnp.concatenate([a0, a0[:, inv_perm]], axis=1)
    B = jnp.concatenate([b0, b0[:, inv_perm]], axis=1)
    bias = jnp.concatenate([bias0, bias0[inv_perm]])[None, :]

    post, expp, nll_parts = pl.pallas_call(
        _gmm_kernel,
        grid=(n_blocks,),
        in_specs=[
            pl.BlockSpec((BLOCK_S, MODEL_DIM), lambda i: (i, 0)),
            pl.BlockSpec((MODEL_DIM, PROJ_DIM), lambda i: (0, 0)),
            pl.BlockSpec((PROJ_DIM, 2 * TC), lambda i: (0, 0)),
            pl.BlockSpec((PROJ_DIM, 2 * TC), lambda i: (0, 0)),
            pl.BlockSpec((1, 2 * TC), lambda i: (0, 0)),
        ],
        out_specs=[
            pl.BlockSpec((BLOCK_S, TC), lambda i: (i, 0)),
            pl.BlockSpec((BLOCK_S, NUM_EXPERTS), lambda i: (i, 0)),
            pl.BlockSpec((1, 1, 128), lambda i: (i, 0, 0)),
        ],
        out_shape=[
            jax.ShapeDtypeStruct((S, TC), jnp.float32),
            jax.ShapeDtypeStruct((S, NUM_EXPERTS), jnp.float32),
            jax.ShapeDtypeStruct((n_blocks, 1, 128), jnp.float32),
        ],
        compiler_params=pltpu.CompilerParams(
            dimension_semantics=("parallel",),
        ),
    )(input, W_proj, A, B, bias)

    nll = -(jnp.sum(nll_parts[:, 0, 0]) / S)
    return (expp, post, nll)
